# baseline (speedup 1.0000x reference)
import functools

import jax
import jax.numpy as jnp
from jax import lax
from jax.experimental import pallas as pl
from jax.experimental.pallas import tpu as pltpu
from jax.experimental.pallas import tpu_sc as plsc

_TEMP = 3.0
_NUM_WORKERS = 32
_CHUNK = 128


def _sc_gather(logits_flat, idx, v_size):
    n_tok, n_half, chunk = idx.shape
    tpw = n_tok // _NUM_WORKERS
    mesh = plsc.VectorSubcoreMesh(core_axis_name="c", subcore_axis_name="s")

    @functools.partial(
        pl.kernel,
        mesh=mesh,
        out_type=jax.ShapeDtypeStruct((n_tok, n_half, chunk), jnp.float32),
        scratch_types=[
            pltpu.VMEM((tpw, n_half, chunk), jnp.int32),
            pltpu.VMEM((tpw, n_half, chunk), jnp.float32),
            pltpu.SemaphoreType.DMA,
        ],
    )
    def gather_kernel(logits_hbm, idx_hbm, out_hbm, idx_v, vals_v, sem):
        wid = lax.axis_index("s") * 2 + lax.axis_index("c")
        base = wid * tpw
        pltpu.sync_copy(idx_hbm.at[pl.ds(base, tpw)], idx_v)

        @pl.loop(0, tpw)
        def _token(t):
            off = (base + t) * v_size

            @pl.loop(0, n_half)
            def _half(h):
                @pl.loop(0, chunk, step=16)
                def _vec(j):
                    sl = idx_v.at[t, h, pl.ds(j, 16)]
                    sl[...] = sl[...] + off

                pltpu.async_copy(
                    logits_hbm.at[idx_v.at[t, h]], vals_v.at[t, h], sem
                )

        pltpu.make_async_copy(out_hbm.at[pl.ds(base, tpw)], vals_v, sem).wait()
        pltpu.sync_copy(vals_v, out_hbm.at[pl.ds(base, tpw)])

    return gather_kernel(logits_flat, idx)


def _tc_loss_body(g_ref, tv_ref, m_ref, out_ref):
    inv_t = 1.0 / _TEMP
    g = g_ref[...] * inv_t
    tv = tv_ref[...] * inv_t
    m_t = jnp.max(tv, axis=-1, keepdims=True)
    e_t = jnp.exp(tv - m_t)
    z_t = jnp.sum(e_t, axis=-1, keepdims=True)
    p_t = e_t / z_t
    logp_t = (tv - m_t) - jnp.log(z_t)
    m_s = jnp.max(g, axis=-1, keepdims=True)
    e_s = jnp.exp(g - m_s)
    lse_s = jnp.log(jnp.sum(e_s, axis=-1, keepdims=True))
    slp = (g - m_s) - lse_s
    kl = jnp.sum(p_t * (logp_t - slp), axis=-1, keepdims=True)
    mf = m_ref[...]
    total = jnp.sum(kl * mf) * (_TEMP * _TEMP)
    cnt = jnp.sum(mf)
    out_ref[...] = (total / jnp.maximum(cnt, 1.0)).reshape(1, 1)


def _tc_loss(gathered, teacher_vals, mask_f):
    return pl.pallas_call(
        _tc_loss_body,
        out_shape=jax.ShapeDtypeStruct((1, 1), jnp.float32),
    )(gathered, teacher_vals, mask_f)


def kernel(student_logits, teacher_vals, teacher_idxs, mask):
    b, s, v_size = student_logits.shape
    k = teacher_vals.shape[-1]
    n = b * s
    logits_flat = student_logits.reshape(n * v_size)
    idx = teacher_idxs.astype(jnp.int32).reshape(n, k // _CHUNK, _CHUNK)
    gathered = _sc_gather(logits_flat, idx, v_size)
    mask_f = mask.reshape(n, 1).astype(jnp.float32)
    out = _tc_loss(
        gathered.reshape(n, k), teacher_vals.reshape(n, k), mask_f
    )
    return out[0, 0]
